# baseline (speedup 1.0000x reference)
import functools

import jax
import jax.numpy as jnp
from jax import lax
from jax.experimental import pallas as pl
from jax.experimental.pallas import tpu as pltpu
from jax.experimental.pallas import tpu_sc as plsc

NC = 2
NS = 16
NW = NC * NS
LANES = 16
RB = 8


def _tanh_via_exp(x):
    return 1.0 - 2.0 / (jnp.exp(2.0 * x) + 1.0)


def kernel(input, emb_table):
    B, S = input.shape
    _, D = emb_table.shape
    nc = D // LANES
    EPW = B // NW

    CH0 = ((S // 2 + 7) // 8) * 8
    CH1 = S - CH0

    def _chunk_max(ref, nrows, acc):
        def body(rb, acc):
            base = rb * RB
            rows = [
                [ref[base + dr, pl.ds(c * LANES, LANES)] for c in range(nc)]
                for dr in range(RB)
            ]
            out = []
            for c in range(nc):
                m = rows[0][c]
                level = [rows[dr][c] for dr in range(RB)]
                while len(level) > 1:
                    nxt = []
                    for j in range(0, len(level) - 1, 2):
                        nxt.append(jnp.maximum(level[j], level[j + 1]))
                    if len(level) % 2:
                        nxt.append(level[-1])
                    level = nxt
                out.append(jnp.maximum(acc[c], level[0]))
            return tuple(out)

        return lax.fori_loop(0, nrows // RB, body, acc)

    mesh = plsc.VectorSubcoreMesh(core_axis_name="c", subcore_axis_name="s")

    @functools.partial(
        pl.kernel,
        out_type=jax.ShapeDtypeStruct((B, D), jnp.float32),
        mesh=mesh,
        compiler_params=pltpu.CompilerParams(use_tc_tiling_on_sc=False),
        scratch_types=[
            pltpu.VMEM((EPW, S), jnp.int32),
            pltpu.VMEM((CH0, D), jnp.float32),
            pltpu.VMEM((CH1, D), jnp.float32),
            pltpu.VMEM((CH0, D), jnp.float32),
            pltpu.VMEM((CH1, D), jnp.float32),
            pltpu.VMEM((EPW, D), jnp.float32),
            pltpu.SemaphoreType.DMA,
            pltpu.SemaphoreType.DMA,
        ],
    )
    def sc_kernel(tbl_hbm, idx_hbm, out_hbm,
                  idx_v, r0a, r0b, r1a, r1b, out_v, sem0, sem1):
        wid = lax.axis_index("s") * NC + lax.axis_index("c")
        base = wid * EPW
        pltpu.sync_copy(idx_hbm.at[pl.ds(base, EPW)], idx_v)

        def fire(i, ra, rb, sem):
            pltpu.async_copy(tbl_hbm.at[idx_v.at[i, pl.ds(0, CH0)]], ra, sem)
            pltpu.async_copy(tbl_hbm.at[idx_v.at[i, pl.ds(CH0, CH1)]], rb, sem)

        def wait_bufs(ra, rb, sem):
            pltpu.make_async_copy(
                tbl_hbm.at[idx_v.at[0, pl.ds(0, CH0)]], ra, sem).wait()
            pltpu.make_async_copy(
                tbl_hbm.at[idx_v.at[0, pl.ds(CH0, CH1)]], rb, sem).wait()

        def consume(i, ra, rb):
            acc = tuple(jnp.full((LANES,), -jnp.inf, jnp.float32)
                        for _ in range(nc))
            acc = _chunk_max(ra, CH0, acc)
            acc = _chunk_max(rb, CH1, acc)
            for c in range(nc):
                out_v[i, pl.ds(c * LANES, LANES)] = _tanh_via_exp(acc[c])

        fire(0, r0a, r0b, sem0)

        @pl.loop(0, EPW, step=2)
        def _(i):
            fire(i + 1, r1a, r1b, sem1)
            wait_bufs(r0a, r0b, sem0)
            consume(i, r0a, r0b)

            @pl.when(i + 2 < EPW)
            def _():
                fire(i + 2, r0a, r0b, sem0)

            wait_bufs(r1a, r1b, sem1)
            consume(i + 1, r1a, r1b)

        pltpu.sync_copy(out_v, out_hbm.at[pl.ds(base, EPW)])

    return sc_kernel(emb_table, input.astype(jnp.int32))


# speedup vs baseline: 1.5247x; 1.5247x over previous
import functools

import jax
import jax.numpy as jnp
from jax import lax
from jax.experimental import pallas as pl
from jax.experimental.layout import Format, Layout, with_layout_constraint
from jax.experimental.pallas import tpu as pltpu
from jax.experimental.pallas import tpu_sc as plsc

NC = 2
NS = 16
NW = NC * NS
LANES = 16
RB = 8


def _tanh_via_exp(x):
    return 1.0 - 2.0 / (jnp.exp(2.0 * x) + 1.0)


def kernel(input, emb_table):
    B, S = input.shape
    _, D = emb_table.shape
    nc = D // LANES
    EPW = B // NW

    CH0 = ((S // 2 + 7) // 8) * 8
    CH1 = S - CH0

    def _chunk_max(ref, nrows, acc):
        def body(rb, acc):
            base = rb * RB
            rows = [
                [ref[base + dr, pl.ds(c * LANES, LANES)] for c in range(nc)]
                for dr in range(RB)
            ]
            out = []
            for c in range(nc):
                m = rows[0][c]
                level = [rows[dr][c] for dr in range(RB)]
                while len(level) > 1:
                    nxt = []
                    for j in range(0, len(level) - 1, 2):
                        nxt.append(jnp.maximum(level[j], level[j + 1]))
                    if len(level) % 2:
                        nxt.append(level[-1])
                    level = nxt
                out.append(jnp.maximum(acc[c], level[0]))
            return tuple(out)

        return lax.fori_loop(0, nrows // RB, body, acc)

    mesh = plsc.VectorSubcoreMesh(core_axis_name="c", subcore_axis_name="s")

    @functools.partial(
        pl.kernel,
        out_type=jax.ShapeDtypeStruct((B, D), jnp.float32),
        mesh=mesh,
        compiler_params=pltpu.CompilerParams(use_tc_tiling_on_sc=False),
        scratch_types=[
            pltpu.VMEM((EPW, S), jnp.int32),
            pltpu.VMEM((CH0, D), jnp.float32),
            pltpu.VMEM((CH1, D), jnp.float32),
            pltpu.VMEM((CH0, D), jnp.float32),
            pltpu.VMEM((CH1, D), jnp.float32),
            pltpu.VMEM((EPW, D), jnp.float32),
            pltpu.SemaphoreType.DMA,
            pltpu.SemaphoreType.DMA,
        ],
    )
    def sc_kernel(tbl_hbm, idx_hbm, out_hbm,
                  idx_v, r0a, r0b, r1a, r1b, out_v, sem0, sem1):
        wid = lax.axis_index("s") * NC + lax.axis_index("c")
        base = wid * EPW
        pltpu.sync_copy(idx_hbm.at[pl.ds(base, EPW)], idx_v)

        def fire(i, ra, rb, sem):
            pltpu.async_copy(tbl_hbm.at[idx_v.at[i, pl.ds(0, CH0)]], ra, sem)
            pltpu.async_copy(tbl_hbm.at[idx_v.at[i, pl.ds(CH0, CH1)]], rb, sem)

        def wait_bufs(ra, rb, sem):
            pltpu.make_async_copy(
                tbl_hbm.at[idx_v.at[0, pl.ds(0, CH0)]], ra, sem).wait()
            pltpu.make_async_copy(
                tbl_hbm.at[idx_v.at[0, pl.ds(CH0, CH1)]], rb, sem).wait()

        def consume(i, ra, rb):
            acc = tuple(jnp.full((LANES,), -jnp.inf, jnp.float32)
                        for _ in range(nc))
            acc = _chunk_max(ra, CH0, acc)
            acc = _chunk_max(rb, CH1, acc)
            for c in range(nc):
                out_v[i, pl.ds(c * LANES, LANES)] = _tanh_via_exp(acc[c])

        fire(0, r0a, r0b, sem0)

        @pl.loop(0, EPW, step=2)
        def _(i):
            fire(i + 1, r1a, r1b, sem1)
            wait_bufs(r0a, r0b, sem0)
            consume(i, r0a, r0b)

            @pl.when(i + 2 < EPW)
            def _():
                fire(i + 2, r0a, r0b, sem0)

            wait_bufs(r1a, r1b, sem1)
            consume(i + 1, r1a, r1b)

        pltpu.sync_copy(out_v, out_hbm.at[pl.ds(base, EPW)])

    tbl = with_layout_constraint(
        emb_table, Layout(major_to_minor=(0, 1), tiling=((8,),)))
    return sc_kernel(tbl, input.astype(jnp.int32))
